# baseline (speedup 1.0000x reference)
import functools

import jax
import jax.numpy as jnp
from jax import lax
from jax.experimental import pallas as pl
from jax.experimental.pallas import tpu as pltpu
from jax.experimental.pallas import tpu_sc as plsc

F32 = jnp.float32
I32 = jnp.int32

B = 2
KN = 12288
BN = 3072
NPTS = 4096
GRID = 64
PLANE = GRID * GRID * GRID
NTILES = 16

_MESH = plsc.VectorSubcoreMesh(core_axis_name="c", subcore_axis_name="s")


def _iota16():
    return lax.iota(I32, 16)


def _corner_idx_weights(crx_ref, cry_ref, crz_ref, vox_ref, wgt_ref, n_groups):
    def body(g, carry):
        ds = pl.ds(g * 16, 16)
        px = crx_ref[ds]
        py = cry_ref[ds]
        pz = crz_ref[ds]
        ix = px * 64.0 - 0.5
        iy = py * 64.0 - 0.5
        iz = pz * 64.0 - 0.5

        def _floor(v):
            ti = v.astype(I32)
            tf = ti.astype(F32)
            i0 = jnp.where(tf > v, ti - 1, ti)
            return i0, i0.astype(F32)

        x0i, x0f = _floor(ix)
        y0i, y0f = _floor(iy)
        z0i, z0f = _floor(iz)
        corner = 0
        for dx in (0, 1):
            xf = x0f + float(dx)
            wx = 1.0 - jnp.abs(ix - xf)
            vx = (xf >= 0.0) & (xf <= 63.0)
            xc = jnp.clip(x0i + dx, 0, 63)
            for dy in (0, 1):
                yf = y0f + float(dy)
                wy = 1.0 - jnp.abs(iy - yf)
                vy = (yf >= 0.0) & (yf <= 63.0)
                yc = jnp.clip(y0i + dy, 0, 63)
                for dz in (0, 1):
                    zf = z0f + float(dz)
                    wz = 1.0 - jnp.abs(iz - zf)
                    vz = (zf >= 0.0) & (zf <= 63.0)
                    zc = jnp.clip(z0i + dz, 0, 63)
                    w = wx * wy * wz
                    valid = vx & vy & vz
                    wv = w * jnp.where(valid, 1.0, 0.0).astype(F32)
                    vox = (zc * 64 + yc) * 64 + xc
                    vox_ref[corner, pl.ds(g * 16, 16)] = vox
                    wgt_ref[corner, pl.ds(g * 16, 16)] = wv
                    corner += 1
        return carry

    lax.fori_loop(0, n_groups, body, 0)


FROWS = 104


@functools.partial(
    pl.kernel,
    mesh=_MESH,
    out_type=[
        jax.ShapeDtypeStruct((B, NPTS), F32),
        jax.ShapeDtypeStruct((B, NPTS), F32),
        jax.ShapeDtypeStruct((B, NPTS), F32),
        jax.ShapeDtypeStruct((FROWS, B * NPTS), F32),
    ],
    scratch_types=[
        pltpu.VMEM((768,), F32),
        pltpu.VMEM((768,), F32),
        pltpu.VMEM((768,), F32),
        pltpu.VMEM((8, 768), I32),
        pltpu.VMEM((8, 768), F32),
        pltpu.VMEM((12 * 8 * 256,), I32),
        pltpu.VMEM((12 * 8 * 256,), F32),
        pltpu.VMEM((768,), I32),
        pltpu.VMEM((768,), I32),
        pltpu.VMEM((768,), I32),
        pltpu.VMEM((768,), I32),
        pltpu.VMEM((256,), I32),
        pltpu.VMEM((16 * 256,), I32),
        pltpu.VMEM((256,), I32),
        pltpu.VMEM((256,), I32),
        pltpu.SMEM((256,), I32),
        pltpu.SMEM((256,), I32),
        pltpu.VMEM((256,), I32),
        pltpu.VMEM((256,), F32),
        pltpu.VMEM((256,), F32),
        pltpu.VMEM((256,), F32),
        pltpu.VMEM((FROWS, 256), F32),
        pltpu.VMEM_SHARED((KN,), I32),
        pltpu.VMEM_SHARED((KN,), I32),
        pltpu.VMEM_SHARED((KN,), I32),
        pltpu.VMEM_SHARED((KN,), I32),
        pltpu.VMEM_SHARED((16 * 256,), I32),
    ],
)
def _sc_render(out_hbm, p1_hbm, p2_hbm, ox_hbm, oy_hbm, oz_hbm,
               cx_hbm, cy_hbm, cz_hbm,
               px_hbm, py_hbm, pz_hbm, feat_hbm,
               crx, cry, crz, vox, wgt, gidx, vals, keys, pay, digits,
               positions, histv, gridv, totv, minev, hist_s, offs_s,
               selid, selx, sely, selz, featv, kcur, knxt, pcur, pnxt, hgrid):
    b = lax.axis_index("c")
    s = lax.axis_index("s")
    npt = KN // NTILES
    base_pt = s * npt
    pltpu.sync_copy(ox_hbm.at[pl.ds(b * KN + base_pt, npt)], crx)
    pltpu.sync_copy(oy_hbm.at[pl.ds(b * KN + base_pt, npt)], cry)
    pltpu.sync_copy(oz_hbm.at[pl.ds(b * KN + base_pt, npt)], crz)
    _corner_idx_weights(crx, cry, crz, vox, wgt, npt // 16)

    def idx_body(g, carry):
        for ch in range(3):
            cb = (b * 3 + ch) * PLANE
            for c in range(8):
                v = vox[c, pl.ds(g * 16, 16)]
                gidx[pl.ds((ch * 8 + c) * npt + g * 16, 16)] = v + cb
        return carry

    lax.fori_loop(0, npt // 16, idx_body, 0)
    pltpu.sync_copy(out_hbm.at[gidx], vals)

    def u_body(g, carry):
        ds = pl.ds(g * 16, 16)
        acc = [jnp.zeros((16,), F32) for _ in range(3)]
        for c in range(8):
            w = wgt[c, ds]
            for ch in range(3):
                v = vals[pl.ds((ch * 8 + c) * npt + g * 16, 16)]
                acc[ch] = acc[ch] + v * w
        a, bb, cc = acc
        mx = jnp.maximum(a, bb)
        mn = jnp.minimum(a, bb)
        v0 = jnp.maximum(mx, cc)
        v1 = jnp.maximum(mn, jnp.minimum(mx, cc))
        u = -1.0 * (v0 - v1)
        ub = lax.bitcast_convert_type(u + 0.0, I32)
        inv = jnp.where(ub < 0, ub, jnp.int32(0x7FFFFFFF) - ub)
        keys[ds] = inv
        pay[ds] = base_pt + g * 16 + _iota16()
        return carry

    lax.fori_loop(0, npt // 16, u_body, 0)
    pltpu.sync_copy(keys, kcur.at[pl.ds(base_pt, npt)])
    pltpu.sync_copy(pay, pcur.at[pl.ds(base_pt, npt)])
    plsc.subcore_barrier()

    for p in range(4):
        src_k, src_p = (kcur, pcur) if p % 2 == 0 else (knxt, pnxt)
        dst_k, dst_p = (knxt, pnxt) if p % 2 == 0 else (kcur, pcur)
        shift = p * 8
        pltpu.sync_copy(src_k.at[pl.ds(base_pt, npt)], keys)
        pltpu.sync_copy(src_p.at[pl.ds(base_pt, npt)], pay)

        def dig_body(g, carry):
            ds = pl.ds(g * 16, 16)
            digits[ds] = lax.shift_right_logical(keys[ds], shift) & 0xFF
            return carry

        lax.fori_loop(0, npt // 16, dig_body, 0)

        def hzero_body(d, carry):
            hist_s[d] = jnp.int32(0)
            return carry

        lax.fori_loop(0, 256, hzero_body, 0)

        def hist_body(g, carry):
            dv = digits[pl.ds(g * 16, 16)]
            for j in range(16):
                d = dv[j]
                hist_s[d] = hist_s[d] + 1
            return carry

        lax.fori_loop(0, npt // 16, hist_body, 0)

        lane = _iota16()

        def hpub_body(g, carry):
            hv = jnp.zeros((16,), I32)
            for j in range(16):
                hv = jnp.where(lane == j, hist_s[g * 16 + j], hv)
            histv[pl.ds(g * 16, 16)] = hv
            return carry

        lax.fori_loop(0, 256 // 16, hpub_body, 0)
        pltpu.sync_copy(histv, hgrid.at[pl.ds(s * 256, 256)])
        plsc.subcore_barrier()
        pltpu.sync_copy(hgrid, gridv)

        def offs_body(g, carry):
            ds = pl.ds(g * 16, 16)
            tot = jnp.zeros((16,), I32)
            mine = jnp.zeros((16,), I32)
            for t in range(NTILES):
                row = gridv[pl.ds(t * 256 + g * 16, 16)]
                tot = tot + row
                mine = mine + jnp.where(t < s, row, 0)
            totv[ds] = tot
            minev[ds] = mine
            return carry

        lax.fori_loop(0, 256 // 16, offs_body, 0)

        def pfx_body(g, acc):
            tv = totv[pl.ds(g * 16, 16)]
            mv = minev[pl.ds(g * 16, 16)]
            for j in range(16):
                offs_s[g * 16 + j] = acc + mv[j]
                acc = acc + tv[j]
            return acc

        lax.fori_loop(0, 256 // 16, pfx_body, jnp.int32(0))

        def rank_body(g, carry):
            dv = digits[pl.ds(g * 16, 16)]
            pv = jnp.zeros((16,), I32)
            for j in range(16):
                d = dv[j]
                p = offs_s[d]
                offs_s[d] = p + 1
                pv = jnp.where(lane == j, p, pv)
            positions[pl.ds(g * 16, 16)] = pv
            return carry

        lax.fori_loop(0, npt // 16, rank_body, 0)
        pltpu.sync_copy(keys, dst_k.at[positions])
        pltpu.sync_copy(pay, dst_p.at[positions])
        plsc.subcore_barrier()

    n_sel = NPTS // NTILES

    @pl.when(s < 12)
    def _importance():
        pltpu.sync_copy(pcur.at[pl.ds(s * n_sel, n_sel)], histv)

        def add_body(g, carry):
            ds = pl.ds(g * 16, 16)
            selid[ds] = histv[ds] + b * KN
            return carry

        lax.fori_loop(0, n_sel // 16, add_body, 0)
        pltpu.sync_copy(ox_hbm.at[selid], selx)
        pltpu.sync_copy(oy_hbm.at[selid], sely)
        pltpu.sync_copy(oz_hbm.at[selid], selz)

    @pl.when(s >= 12)
    def _coverage():
        cb = b * (NPTS - BN) + (s - 12) * n_sel
        pltpu.sync_copy(cx_hbm.at[pl.ds(cb, n_sel)], selx)
        pltpu.sync_copy(cy_hbm.at[pl.ds(cb, n_sel)], sely)
        pltpu.sync_copy(cz_hbm.at[pl.ds(cb, n_sel)], selz)

    pltpu.sync_copy(selx, px_hbm.at[b, pl.ds(s * n_sel, n_sel)])
    pltpu.sync_copy(sely, py_hbm.at[b, pl.ds(s * n_sel, n_sel)])
    pltpu.sync_copy(selz, pz_hbm.at[b, pl.ds(s * n_sel, n_sel)])

    _corner_idx_weights(selx, sely, selz, vox, wgt, n_sel // 16)

    _feat_chunk(out_hbm, vox, wgt, featv, b * 3, 3, 0, gidx, vals, n_sel)

    def fine1_body(k, carry):
        _feat_chunk(p1_hbm, vox, wgt, featv, b * 48 + k * 12, 12, 3 + k * 12,
                    gidx, vals, n_sel)
        return carry

    lax.fori_loop(0, 4, fine1_body, 0)

    def fine2_body(k, carry):
        _feat_chunk(p2_hbm, vox, wgt, featv, b * 48 + k * 12, 12, 51 + k * 12,
                    gidx, vals, n_sel)
        return carry

    lax.fori_loop(0, 4, fine2_body, 0)

    def zero_body(g, carry):
        z = jnp.zeros((16,), F32)
        for r in range(99, FROWS):
            featv[r, pl.ds(g * 16, 16)] = z
        return carry

    lax.fori_loop(0, n_sel // 16, zero_body, 0)

    col0 = b * NPTS + s * n_sel
    pltpu.sync_copy(featv, feat_hbm.at[:, pl.ds(col0, n_sel)])


def _feat_chunk(vol_hbm, vox, wgt, featv, ch_base, nch, row_base, idx_ref,
                val_ref, npt):
    def idx_body(g, carry):
        for ch in range(nch):
            cb = (ch_base + ch) * PLANE
            for c in range(8):
                v = vox[c, pl.ds(g * 16, 16)]
                idx_ref[pl.ds((ch * 8 + c) * npt + g * 16, 16)] = v + cb
        return carry

    lax.fori_loop(0, npt // 16, idx_body, 0)
    pltpu.sync_copy(vol_hbm.at[idx_ref], val_ref)

    def acc_body(g, carry):
        ds = pl.ds(g * 16, 16)
        ws = [wgt[c, ds] for c in range(8)]
        for ch in range(nch):
            acc = jnp.zeros((16,), F32)
            for c in range(8):
                v = val_ref[pl.ds((ch * 8 + c) * npt + g * 16, 16)]
                acc = acc + v * ws[c]
            featv[row_base + ch, ds] = acc
        return carry

    lax.fori_loop(0, npt // 16, acc_body, 0)


def _mlp_body(f_ref, w1_ref, b1_ref, w2_ref, b2_ref, w3_ref, b3_ref, o_ref):
    h = (jnp.dot(w1_ref[...], f_ref[...], preferred_element_type=F32)
         + b1_ref[...])
    h = jnp.maximum(h, 0.0)
    h = jnp.dot(w2_ref[...], h, preferred_element_type=F32) + b2_ref[...]
    h = jnp.maximum(h, 0.0)
    o_ref[...] = jnp.dot(w3_ref[...], h, preferred_element_type=F32) + b3_ref[...]


def _mlp(f, W1p, b1c, W2, b2c, W3p, b3c):
    return pl.pallas_call(
        _mlp_body,
        out_shape=jax.ShapeDtypeStruct((8, B * NPTS), F32),
    )(f, W1p, b1c, W2, b2c, W3p, b3c)


def kernel(x, p2_1, p2_2, out, W1, b1, W2, b2, W3, b3):
    del x
    key = jax.random.key(42)
    k1, k2 = jax.random.split(key)
    over = jax.random.uniform(k1, (B, KN, 3), dtype=out.dtype)
    coverage = jax.random.uniform(k2, (B, NPTS - BN, 3), dtype=out.dtype)

    out_flat = out.reshape(B * 3 * PLANE)
    p1_flat = p2_1.reshape(B * 48 * PLANE)
    p2_flat = p2_2.reshape(B * 48 * PLANE)

    px, py, pz, feat = _sc_render(
        out_flat, p1_flat, p2_flat,
        over[:, :, 0].reshape(B * KN), over[:, :, 1].reshape(B * KN),
        over[:, :, 2].reshape(B * KN),
        coverage[:, :, 0].reshape(-1), coverage[:, :, 1].reshape(-1),
        coverage[:, :, 2].reshape(-1))
    points = jnp.stack([px, py, pz], axis=-1)

    W1p = jnp.pad(W1, ((0, 0), (0, FROWS - 99)))
    W3p = jnp.pad(W3, ((0, 5), (0, 0)))
    y = _mlp(feat, W1p, b1.reshape(64, 1), W2,
             b2.reshape(32, 1), W3p, jnp.pad(b3, (0, 5)).reshape(8, 1))
    rend = y[:3].reshape(3, B, NPTS).transpose(1, 0, 2)
    return rend, points


# speedup vs baseline: 1.3723x; 1.3723x over previous
import functools

import jax
import jax.numpy as jnp
from jax import lax
from jax.experimental import pallas as pl
from jax.experimental.pallas import tpu as pltpu
from jax.experimental.pallas import tpu_sc as plsc

F32 = jnp.float32
I32 = jnp.int32

B = 2
KN = 12288
BN = 3072
NPTS = 4096
GRID = 64
PLANE = GRID * GRID * GRID
NTILES = 16

_MESH = plsc.VectorSubcoreMesh(core_axis_name="c", subcore_axis_name="s")


def _iota16():
    return lax.iota(I32, 16)


def _corner_idx_weights(crx_ref, cry_ref, crz_ref, vox_ref, wgt_ref, n_groups):
    def body(g, carry):
        ds = pl.ds(g * 16, 16)
        px = crx_ref[ds]
        py = cry_ref[ds]
        pz = crz_ref[ds]
        ix = px * 64.0 - 0.5
        iy = py * 64.0 - 0.5
        iz = pz * 64.0 - 0.5

        def _floor(v):
            ti = v.astype(I32)
            tf = ti.astype(F32)
            i0 = jnp.where(tf > v, ti - 1, ti)
            return i0, i0.astype(F32)

        x0i, x0f = _floor(ix)
        y0i, y0f = _floor(iy)
        z0i, z0f = _floor(iz)
        corner = 0
        for dx in (0, 1):
            xf = x0f + float(dx)
            wx = 1.0 - jnp.abs(ix - xf)
            vx = (xf >= 0.0) & (xf <= 63.0)
            xc = jnp.clip(x0i + dx, 0, 63)
            for dy in (0, 1):
                yf = y0f + float(dy)
                wy = 1.0 - jnp.abs(iy - yf)
                vy = (yf >= 0.0) & (yf <= 63.0)
                yc = jnp.clip(y0i + dy, 0, 63)
                for dz in (0, 1):
                    zf = z0f + float(dz)
                    wz = 1.0 - jnp.abs(iz - zf)
                    vz = (zf >= 0.0) & (zf <= 63.0)
                    zc = jnp.clip(z0i + dz, 0, 63)
                    w = wx * wy * wz
                    valid = vx & vy & vz
                    wv = w * jnp.where(valid, 1.0, 0.0).astype(F32)
                    vox = (zc * 64 + yc) * 64 + xc
                    vox_ref[corner, pl.ds(g * 16, 16)] = vox
                    wgt_ref[corner, pl.ds(g * 16, 16)] = wv
                    corner += 1
        return carry

    lax.fori_loop(0, n_groups, body, 0)


@functools.partial(
    pl.kernel,
    mesh=_MESH,
    out_type=[
        jax.ShapeDtypeStruct((B, NPTS), F32),
        jax.ShapeDtypeStruct((B, NPTS), F32),
        jax.ShapeDtypeStruct((B, NPTS), F32),
    ],
    scratch_types=[
        pltpu.VMEM((768,), F32),
        pltpu.VMEM((768,), F32),
        pltpu.VMEM((768,), F32),
        pltpu.VMEM((8, 768), I32),
        pltpu.VMEM((8, 768), F32),
        pltpu.VMEM((3 * 8 * 768,), I32),
        pltpu.VMEM((3 * 8 * 768,), F32),
        pltpu.VMEM((768,), I32),
        pltpu.VMEM((768,), I32),
        pltpu.VMEM((768,), I32),
        pltpu.VMEM((768,), I32),
        pltpu.VMEM((256,), I32),
        pltpu.VMEM((16 * 256,), I32),
        pltpu.VMEM((256,), I32),
        pltpu.VMEM((256,), I32),
        pltpu.SMEM((256,), I32),
        pltpu.SMEM((256,), I32),
        pltpu.VMEM((256,), I32),
        pltpu.VMEM((256,), F32),
        pltpu.VMEM((256,), F32),
        pltpu.VMEM((256,), F32),
        pltpu.VMEM_SHARED((KN,), I32),
        pltpu.VMEM_SHARED((KN,), I32),
        pltpu.VMEM_SHARED((KN,), I32),
        pltpu.VMEM_SHARED((KN,), I32),
        pltpu.VMEM_SHARED((16 * 256,), I32),
    ],
)
def _sc_select(out_hbm, ox_hbm, oy_hbm, oz_hbm, cx_hbm, cy_hbm, cz_hbm,
               px_hbm, py_hbm, pz_hbm,
               crx, cry, crz, vox, wgt, gidx, vals, keys, pay, digits,
               positions, histv, gridv, totv, minev, hist_s, offs_s,
               selid, selx, sely, selz, kcur, knxt, pcur, pnxt, hgrid):
    b = lax.axis_index("c")
    s = lax.axis_index("s")
    npt = KN // NTILES
    base_pt = s * npt
    pltpu.sync_copy(ox_hbm.at[pl.ds(b * KN + base_pt, npt)], crx)
    pltpu.sync_copy(oy_hbm.at[pl.ds(b * KN + base_pt, npt)], cry)
    pltpu.sync_copy(oz_hbm.at[pl.ds(b * KN + base_pt, npt)], crz)
    _corner_idx_weights(crx, cry, crz, vox, wgt, npt // 16)

    def idx_body(g, carry):
        for ch in range(3):
            cb = (b * 3 + ch) * PLANE
            for c in range(8):
                v = vox[c, pl.ds(g * 16, 16)]
                gidx[pl.ds((ch * 8 + c) * npt + g * 16, 16)] = v + cb
        return carry

    lax.fori_loop(0, npt // 16, idx_body, 0)
    pltpu.sync_copy(out_hbm.at[gidx], vals)

    def u_body(g, carry):
        ds = pl.ds(g * 16, 16)
        acc = [jnp.zeros((16,), F32) for _ in range(3)]
        for c in range(8):
            w = wgt[c, ds]
            for ch in range(3):
                v = vals[pl.ds((ch * 8 + c) * npt + g * 16, 16)]
                acc[ch] = acc[ch] + v * w
        a, bb, cc = acc
        mx = jnp.maximum(a, bb)
        mn = jnp.minimum(a, bb)
        v0 = jnp.maximum(mx, cc)
        v1 = jnp.maximum(mn, jnp.minimum(mx, cc))
        u = -1.0 * (v0 - v1)
        ub = lax.bitcast_convert_type(u + 0.0, I32)
        inv = jnp.where(ub < 0, ub, jnp.int32(0x7FFFFFFF) - ub)
        keys[ds] = inv
        pay[ds] = base_pt + g * 16 + _iota16()
        return carry

    lax.fori_loop(0, npt // 16, u_body, 0)
    pltpu.sync_copy(keys, kcur.at[pl.ds(base_pt, npt)])
    pltpu.sync_copy(pay, pcur.at[pl.ds(base_pt, npt)])
    plsc.subcore_barrier()

    for p in range(4):
        src_k, src_p = (kcur, pcur) if p % 2 == 0 else (knxt, pnxt)
        dst_k, dst_p = (knxt, pnxt) if p % 2 == 0 else (kcur, pcur)
        shift = p * 8
        pltpu.sync_copy(src_k.at[pl.ds(base_pt, npt)], keys)
        pltpu.sync_copy(src_p.at[pl.ds(base_pt, npt)], pay)

        def dig_body(g, carry):
            ds = pl.ds(g * 16, 16)
            digits[ds] = lax.shift_right_logical(keys[ds], shift) & 0xFF
            return carry

        lax.fori_loop(0, npt // 16, dig_body, 0)

        def hzero_body(d, carry):
            hist_s[d] = jnp.int32(0)
            return carry

        lax.fori_loop(0, 256, hzero_body, 0)

        def hist_body(g, carry):
            dv = digits[pl.ds(g * 16, 16)]
            for j in range(16):
                d = dv[j]
                hist_s[d] = hist_s[d] + 1
            return carry

        lax.fori_loop(0, npt // 16, hist_body, 0)

        lane = _iota16()

        def hpub_body(g, carry):
            hv = jnp.zeros((16,), I32)
            for j in range(16):
                hv = jnp.where(lane == j, hist_s[g * 16 + j], hv)
            histv[pl.ds(g * 16, 16)] = hv
            return carry

        lax.fori_loop(0, 256 // 16, hpub_body, 0)
        pltpu.sync_copy(histv, hgrid.at[pl.ds(s * 256, 256)])
        plsc.subcore_barrier()
        pltpu.sync_copy(hgrid, gridv)

        def offs_body(g, carry):
            ds = pl.ds(g * 16, 16)
            tot = jnp.zeros((16,), I32)
            mine = jnp.zeros((16,), I32)
            for t in range(NTILES):
                row = gridv[pl.ds(t * 256 + g * 16, 16)]
                tot = tot + row
                mine = mine + jnp.where(t < s, row, 0)
            totv[ds] = tot
            minev[ds] = mine
            return carry

        lax.fori_loop(0, 256 // 16, offs_body, 0)

        def pfx_body(g, acc):
            tv = totv[pl.ds(g * 16, 16)]
            mv = minev[pl.ds(g * 16, 16)]
            for j in range(16):
                offs_s[g * 16 + j] = acc + mv[j]
                acc = acc + tv[j]
            return acc

        lax.fori_loop(0, 256 // 16, pfx_body, jnp.int32(0))

        def rank_body(g, carry):
            dv = digits[pl.ds(g * 16, 16)]
            pv = jnp.zeros((16,), I32)
            for j in range(16):
                d = dv[j]
                p = offs_s[d]
                offs_s[d] = p + 1
                pv = jnp.where(lane == j, p, pv)
            positions[pl.ds(g * 16, 16)] = pv
            return carry

        lax.fori_loop(0, npt // 16, rank_body, 0)
        pltpu.sync_copy(keys, dst_k.at[positions])
        pltpu.sync_copy(pay, dst_p.at[positions])
        plsc.subcore_barrier()

    n_sel = NPTS // NTILES

    @pl.when(s < 12)
    def _importance():
        pltpu.sync_copy(pcur.at[pl.ds(s * n_sel, n_sel)], histv)

        def add_body(g, carry):
            ds = pl.ds(g * 16, 16)
            selid[ds] = histv[ds] + b * KN
            return carry

        lax.fori_loop(0, n_sel // 16, add_body, 0)
        pltpu.sync_copy(ox_hbm.at[selid], selx)
        pltpu.sync_copy(oy_hbm.at[selid], sely)
        pltpu.sync_copy(oz_hbm.at[selid], selz)

    @pl.when(s >= 12)
    def _coverage():
        cb = b * (NPTS - BN) + (s - 12) * n_sel
        pltpu.sync_copy(cx_hbm.at[pl.ds(cb, n_sel)], selx)
        pltpu.sync_copy(cy_hbm.at[pl.ds(cb, n_sel)], sely)
        pltpu.sync_copy(cz_hbm.at[pl.ds(cb, n_sel)], selz)

    pltpu.sync_copy(selx, px_hbm.at[b, pl.ds(s * n_sel, n_sel)])
    pltpu.sync_copy(sely, py_hbm.at[b, pl.ds(s * n_sel, n_sel)])
    pltpu.sync_copy(selz, pz_hbm.at[b, pl.ds(s * n_sel, n_sel)])


def _feat_common(px_hbm, py_hbm, pz_hbm, crx, cry, crz, vox, wgt):
    b = lax.axis_index("c")
    s = lax.axis_index("s")
    npt = NPTS // NTILES
    base_pt = s * npt
    pltpu.sync_copy(px_hbm.at[b, pl.ds(base_pt, npt)], crx)
    pltpu.sync_copy(py_hbm.at[b, pl.ds(base_pt, npt)], cry)
    pltpu.sync_copy(pz_hbm.at[b, pl.ds(base_pt, npt)], crz)
    _corner_idx_weights(crx, cry, crz, vox, wgt, npt // 16)
    return b, s, npt


def _feat_chunk(vol_hbm, vox, wgt, featv, ch_base, nch, row_base, idx_ref,
                val_ref, npt):
    def idx_body(g, carry):
        for ch in range(nch):
            cb = (ch_base + ch) * PLANE
            for c in range(8):
                v = vox[c, pl.ds(g * 16, 16)]
                idx_ref[pl.ds((ch * 8 + c) * npt + g * 16, 16)] = v + cb
        return carry

    lax.fori_loop(0, npt // 16, idx_body, 0)
    pltpu.sync_copy(vol_hbm.at[idx_ref], val_ref)

    def acc_body(g, carry):
        ds = pl.ds(g * 16, 16)
        ws = [wgt[c, ds] for c in range(8)]
        for ch in range(nch):
            acc = jnp.zeros((16,), F32)
            for c in range(8):
                v = val_ref[pl.ds((ch * 8 + c) * npt + g * 16, 16)]
                acc = acc + v * ws[c]
            featv[row_base + ch, ds] = acc
        return carry

    lax.fori_loop(0, npt // 16, acc_body, 0)


@functools.partial(
    pl.kernel,
    mesh=_MESH,
    out_type=jax.ShapeDtypeStruct((56, B * NPTS), F32),
    scratch_types=[
        pltpu.VMEM((256,), F32),
        pltpu.VMEM((256,), F32),
        pltpu.VMEM((256,), F32),
        pltpu.VMEM((8, 256), I32),
        pltpu.VMEM((8, 256), F32),
        pltpu.VMEM((12 * 8 * 256,), I32),
        pltpu.VMEM((12 * 8 * 256,), F32),
        pltpu.VMEM((3 * 8 * 256,), I32),
        pltpu.VMEM((3 * 8 * 256,), F32),
        pltpu.VMEM((56, 256), F32),
    ],
)
def _sc_feat_a(out_hbm, p1_hbm, px_hbm, py_hbm, pz_hbm, feat_hbm, crx, cry,
               crz, vox, wgt, gidx, vals, gidx3, vals3, featv):
    b, s, npt = _feat_common(px_hbm, py_hbm, pz_hbm, crx, cry, crz, vox, wgt)

    _feat_chunk(out_hbm, vox, wgt, featv, b * 3, 3, 0, gidx3, vals3, npt)

    def fine_body(k, carry):
        _feat_chunk(p1_hbm, vox, wgt, featv, b * 48 + k * 12, 12, 3 + k * 12,
                    gidx, vals, npt)
        return carry

    lax.fori_loop(0, 4, fine_body, 0)

    def zero_body(g, carry):
        z = jnp.zeros((16,), F32)
        for r in range(51, 56):
            featv[r, pl.ds(g * 16, 16)] = z
        return carry

    lax.fori_loop(0, npt // 16, zero_body, 0)

    col0 = b * NPTS + s * npt
    pltpu.sync_copy(featv, feat_hbm.at[:, pl.ds(col0, npt)])


@functools.partial(
    pl.kernel,
    mesh=_MESH,
    out_type=jax.ShapeDtypeStruct((48, B * NPTS), F32),
    scratch_types=[
        pltpu.VMEM((256,), F32),
        pltpu.VMEM((256,), F32),
        pltpu.VMEM((256,), F32),
        pltpu.VMEM((8, 256), I32),
        pltpu.VMEM((8, 256), F32),
        pltpu.VMEM((12 * 8 * 256,), I32),
        pltpu.VMEM((12 * 8 * 256,), F32),
        pltpu.VMEM((48, 256), F32),
    ],
)
def _sc_feat_b(p2_hbm, px_hbm, py_hbm, pz_hbm, feat_hbm, crx, cry, crz, vox,
               wgt, gidx, vals, featv):
    b, s, npt = _feat_common(px_hbm, py_hbm, pz_hbm, crx, cry, crz, vox, wgt)

    def fine_body(k, carry):
        _feat_chunk(p2_hbm, vox, wgt, featv, b * 48 + k * 12, 12, k * 12,
                    gidx, vals, npt)
        return carry

    lax.fori_loop(0, 4, fine_body, 0)

    col0 = b * NPTS + s * npt
    pltpu.sync_copy(featv, feat_hbm.at[:, pl.ds(col0, npt)])


def _mlp_body(fa_ref, fb_ref, w1a_ref, w1b_ref, b1_ref, w2_ref, b2_ref,
              w3_ref, b3_ref, o_ref):
    h = (jnp.dot(w1a_ref[...], fa_ref[...], preferred_element_type=F32)
         + jnp.dot(w1b_ref[...], fb_ref[...], preferred_element_type=F32)
         + b1_ref[...])
    h = jnp.maximum(h, 0.0)
    h = jnp.dot(w2_ref[...], h, preferred_element_type=F32) + b2_ref[...]
    h = jnp.maximum(h, 0.0)
    o_ref[...] = jnp.dot(w3_ref[...], h, preferred_element_type=F32) + b3_ref[...]


def _mlp(fa, fb, W1a, W1b, b1c, W2, b2c, W3p, b3c):
    return pl.pallas_call(
        _mlp_body,
        out_shape=jax.ShapeDtypeStruct((8, B * NPTS), F32),
    )(fa, fb, W1a, W1b, b1c, W2, b2c, W3p, b3c)


def kernel(x, p2_1, p2_2, out, W1, b1, W2, b2, W3, b3):
    del x
    key = jax.random.key(42)
    k1, k2 = jax.random.split(key)
    over = jax.random.uniform(k1, (B, KN, 3), dtype=out.dtype)
    coverage = jax.random.uniform(k2, (B, NPTS - BN, 3), dtype=out.dtype)

    out_flat = out.reshape(B * 3 * PLANE)
    p1_flat = p2_1.reshape(B * 48 * PLANE)
    p2_flat = p2_2.reshape(B * 48 * PLANE)

    px, py, pz = _sc_select(
        out_flat,
        over[:, :, 0].reshape(B * KN), over[:, :, 1].reshape(B * KN),
        over[:, :, 2].reshape(B * KN),
        coverage[:, :, 0].reshape(-1), coverage[:, :, 1].reshape(-1),
        coverage[:, :, 2].reshape(-1))
    points = jnp.stack([px, py, pz], axis=-1)

    feat_a = _sc_feat_a(out_flat, p1_flat, px, py, pz)
    feat_b = _sc_feat_b(p2_flat, px, py, pz)

    W1a = jnp.pad(W1[:, :51], ((0, 0), (0, 5)))
    W1b = W1[:, 51:99]
    W3p = jnp.pad(W3, ((0, 5), (0, 0)))
    y = _mlp(feat_a, feat_b, W1a, W1b, b1.reshape(64, 1), W2,
             b2.reshape(32, 1), W3p, jnp.pad(b3, (0, 5)).reshape(8, 1))
    rend = y[:3].reshape(3, B, NPTS).transpose(1, 0, 2)
    return rend, points


# speedup vs baseline: 1.3737x; 1.0011x over previous
import functools

import jax
import jax.numpy as jnp
from jax import lax
from jax.experimental import pallas as pl
from jax.experimental.pallas import tpu as pltpu
from jax.experimental.pallas import tpu_sc as plsc

F32 = jnp.float32
I32 = jnp.int32

B = 2
KN = 12288
BN = 3072
NPTS = 4096
GRID = 64
PLANE = GRID * GRID * GRID
NTILES = 16

_MESH = plsc.VectorSubcoreMesh(core_axis_name="c", subcore_axis_name="s")


def _iota16():
    return lax.iota(I32, 16)


def _corner_idx_weights(crx_ref, cry_ref, crz_ref, vox_ref, wgt_ref, n_groups):
    def body(g, carry):
        ds = pl.ds(g * 16, 16)
        px = crx_ref[ds]
        py = cry_ref[ds]
        pz = crz_ref[ds]
        ix = px * 64.0 - 0.5
        iy = py * 64.0 - 0.5
        iz = pz * 64.0 - 0.5

        def _floor(v):
            ti = v.astype(I32)
            tf = ti.astype(F32)
            i0 = jnp.where(tf > v, ti - 1, ti)
            return i0, i0.astype(F32)

        x0i, x0f = _floor(ix)
        y0i, y0f = _floor(iy)
        z0i, z0f = _floor(iz)
        corner = 0
        for dx in (0, 1):
            xf = x0f + float(dx)
            wx = 1.0 - jnp.abs(ix - xf)
            vx = (xf >= 0.0) & (xf <= 63.0)
            xc = jnp.clip(x0i + dx, 0, 63)
            for dy in (0, 1):
                yf = y0f + float(dy)
                wy = 1.0 - jnp.abs(iy - yf)
                vy = (yf >= 0.0) & (yf <= 63.0)
                yc = jnp.clip(y0i + dy, 0, 63)
                for dz in (0, 1):
                    zf = z0f + float(dz)
                    wz = 1.0 - jnp.abs(iz - zf)
                    vz = (zf >= 0.0) & (zf <= 63.0)
                    zc = jnp.clip(z0i + dz, 0, 63)
                    w = wx * wy * wz
                    valid = vx & vy & vz
                    wv = w * jnp.where(valid, 1.0, 0.0).astype(F32)
                    vox = (zc * 64 + yc) * 64 + xc
                    vox_ref[corner, pl.ds(g * 16, 16)] = vox
                    wgt_ref[corner, pl.ds(g * 16, 16)] = wv
                    corner += 1
        return carry

    lax.fori_loop(0, n_groups, body, 0)


@functools.partial(
    pl.kernel,
    mesh=_MESH,
    out_type=[
        jax.ShapeDtypeStruct((B, NPTS), F32),
        jax.ShapeDtypeStruct((B, NPTS), F32),
        jax.ShapeDtypeStruct((B, NPTS), F32),
    ],
    scratch_types=[
        pltpu.VMEM((768,), F32),
        pltpu.VMEM((768,), F32),
        pltpu.VMEM((768,), F32),
        pltpu.VMEM((8, 768), I32),
        pltpu.VMEM((8, 768), F32),
        pltpu.VMEM((3 * 8 * 768,), I32),
        pltpu.VMEM((3 * 8 * 768,), F32),
        pltpu.VMEM((768,), I32),
        pltpu.VMEM((768,), I32),
        pltpu.VMEM((768,), I32),
        pltpu.VMEM((768,), I32),
        pltpu.VMEM((256,), I32),
        pltpu.VMEM((16 * 256,), I32),
        pltpu.VMEM((256,), I32),
        pltpu.VMEM((256,), I32),
        pltpu.SMEM((256,), I32),
        pltpu.SMEM((256,), I32),
        pltpu.VMEM((256,), I32),
        pltpu.VMEM((256,), F32),
        pltpu.VMEM((256,), F32),
        pltpu.VMEM((256,), F32),
        pltpu.VMEM_SHARED((KN,), I32),
        pltpu.VMEM_SHARED((KN,), I32),
        pltpu.VMEM_SHARED((KN,), I32),
        pltpu.VMEM_SHARED((KN,), I32),
        pltpu.VMEM_SHARED((16 * 256,), I32),
    ],
)
def _sc_select(out_hbm, ox_hbm, oy_hbm, oz_hbm, cx_hbm, cy_hbm, cz_hbm,
               px_hbm, py_hbm, pz_hbm,
               crx, cry, crz, vox, wgt, gidx, vals, keys, pay, digits,
               positions, histv, gridv, totv, minev, hist_s, offs_s,
               selid, selx, sely, selz, kcur, knxt, pcur, pnxt, hgrid):
    b = lax.axis_index("c")
    s = lax.axis_index("s")
    npt = KN // NTILES
    base_pt = s * npt
    pltpu.sync_copy(ox_hbm.at[pl.ds(b * KN + base_pt, npt)], crx)
    pltpu.sync_copy(oy_hbm.at[pl.ds(b * KN + base_pt, npt)], cry)
    pltpu.sync_copy(oz_hbm.at[pl.ds(b * KN + base_pt, npt)], crz)
    _corner_idx_weights(crx, cry, crz, vox, wgt, npt // 16)

    def idx_body(g, carry):
        for ch in range(3):
            cb = (b * 3 + ch) * PLANE
            for c in range(8):
                v = vox[c, pl.ds(g * 16, 16)]
                gidx[pl.ds((ch * 8 + c) * npt + g * 16, 16)] = v + cb
        return carry

    lax.fori_loop(0, npt // 16, idx_body, 0)
    pltpu.sync_copy(out_hbm.at[gidx], vals)

    def u_body(g, carry):
        ds = pl.ds(g * 16, 16)
        acc = [jnp.zeros((16,), F32) for _ in range(3)]
        for c in range(8):
            w = wgt[c, ds]
            for ch in range(3):
                v = vals[pl.ds((ch * 8 + c) * npt + g * 16, 16)]
                acc[ch] = acc[ch] + v * w
        a, bb, cc = acc
        mx = jnp.maximum(a, bb)
        mn = jnp.minimum(a, bb)
        v0 = jnp.maximum(mx, cc)
        v1 = jnp.maximum(mn, jnp.minimum(mx, cc))
        u = -1.0 * (v0 - v1)
        ub = lax.bitcast_convert_type(u + 0.0, I32)
        inv = jnp.where(ub < 0, ub, jnp.int32(-0x7FFFFFFF - 1))
        keys[ds] = inv
        pay[ds] = base_pt + g * 16 + _iota16()
        return carry

    lax.fori_loop(0, npt // 16, u_body, 0)
    pltpu.sync_copy(keys, kcur.at[pl.ds(base_pt, npt)])
    pltpu.sync_copy(pay, pcur.at[pl.ds(base_pt, npt)])
    plsc.subcore_barrier()

    for p in range(4):
        src_k, src_p = (kcur, pcur) if p % 2 == 0 else (knxt, pnxt)
        dst_k, dst_p = (knxt, pnxt) if p % 2 == 0 else (kcur, pcur)
        shift = p * 8
        pltpu.sync_copy(src_k.at[pl.ds(base_pt, npt)], keys)
        pltpu.sync_copy(src_p.at[pl.ds(base_pt, npt)], pay)

        def dig_body(g, carry):
            ds = pl.ds(g * 16, 16)
            digits[ds] = lax.shift_right_logical(keys[ds], shift) & 0xFF
            return carry

        lax.fori_loop(0, npt // 16, dig_body, 0)

        def hzero_body(d, carry):
            hist_s[d] = jnp.int32(0)
            return carry

        lax.fori_loop(0, 256, hzero_body, 0)

        def hist_body(g, carry):
            dv = digits[pl.ds(g * 16, 16)]
            for j in range(16):
                d = dv[j]
                hist_s[d] = hist_s[d] + 1
            return carry

        lax.fori_loop(0, npt // 16, hist_body, 0)

        lane = _iota16()

        def hpub_body(g, carry):
            hv = jnp.zeros((16,), I32)
            for j in range(16):
                hv = jnp.where(lane == j, hist_s[g * 16 + j], hv)
            histv[pl.ds(g * 16, 16)] = hv
            return carry

        lax.fori_loop(0, 256 // 16, hpub_body, 0)
        pltpu.sync_copy(histv, hgrid.at[pl.ds(s * 256, 256)])
        plsc.subcore_barrier()
        pltpu.sync_copy(hgrid, gridv)

        def offs_body(g, carry):
            ds = pl.ds(g * 16, 16)
            tot = jnp.zeros((16,), I32)
            mine = jnp.zeros((16,), I32)
            for t in range(NTILES):
                row = gridv[pl.ds(t * 256 + g * 16, 16)]
                tot = tot + row
                mine = mine + jnp.where(t < s, row, 0)
            totv[ds] = tot
            minev[ds] = mine
            return carry

        lax.fori_loop(0, 256 // 16, offs_body, 0)

        def pfx_body(g, acc):
            tv = totv[pl.ds(g * 16, 16)]
            mv = minev[pl.ds(g * 16, 16)]
            for j in range(16):
                offs_s[g * 16 + j] = acc + mv[j]
                acc = acc + tv[j]
            return acc

        lax.fori_loop(0, 256 // 16, pfx_body, jnp.int32(0))

        def rank_body(g, carry):
            dv = digits[pl.ds(g * 16, 16)]
            pv = jnp.zeros((16,), I32)
            for j in range(16):
                d = dv[j]
                p = offs_s[d]
                offs_s[d] = p + 1
                pv = jnp.where(lane == j, p, pv)
            positions[pl.ds(g * 16, 16)] = pv
            return carry

        lax.fori_loop(0, npt // 16, rank_body, 0)
        pltpu.sync_copy(keys, dst_k.at[positions])
        pltpu.sync_copy(pay, dst_p.at[positions])
        plsc.subcore_barrier()

    n_sel = NPTS // NTILES

    @pl.when(s < 12)
    def _importance():
        pltpu.sync_copy(pcur.at[pl.ds(s * n_sel, n_sel)], histv)

        def add_body(g, carry):
            ds = pl.ds(g * 16, 16)
            selid[ds] = histv[ds] + b * KN
            return carry

        lax.fori_loop(0, n_sel // 16, add_body, 0)
        pltpu.sync_copy(ox_hbm.at[selid], selx)
        pltpu.sync_copy(oy_hbm.at[selid], sely)
        pltpu.sync_copy(oz_hbm.at[selid], selz)

    @pl.when(s >= 12)
    def _coverage():
        cb = b * (NPTS - BN) + (s - 12) * n_sel
        pltpu.sync_copy(cx_hbm.at[pl.ds(cb, n_sel)], selx)
        pltpu.sync_copy(cy_hbm.at[pl.ds(cb, n_sel)], sely)
        pltpu.sync_copy(cz_hbm.at[pl.ds(cb, n_sel)], selz)

    pltpu.sync_copy(selx, px_hbm.at[b, pl.ds(s * n_sel, n_sel)])
    pltpu.sync_copy(sely, py_hbm.at[b, pl.ds(s * n_sel, n_sel)])
    pltpu.sync_copy(selz, pz_hbm.at[b, pl.ds(s * n_sel, n_sel)])


def _feat_common(px_hbm, py_hbm, pz_hbm, crx, cry, crz, vox, wgt):
    b = lax.axis_index("c")
    s = lax.axis_index("s")
    npt = NPTS // NTILES
    base_pt = s * npt
    pltpu.sync_copy(px_hbm.at[b, pl.ds(base_pt, npt)], crx)
    pltpu.sync_copy(py_hbm.at[b, pl.ds(base_pt, npt)], cry)
    pltpu.sync_copy(pz_hbm.at[b, pl.ds(base_pt, npt)], crz)
    _corner_idx_weights(crx, cry, crz, vox, wgt, npt // 16)
    return b, s, npt


def _feat_chunk(vol_hbm, vox, wgt, featv, ch_base, nch, row_base, idx_ref,
                val_ref, npt):
    def idx_body(g, carry):
        for ch in range(nch):
            cb = (ch_base + ch) * PLANE
            for c in range(8):
                v = vox[c, pl.ds(g * 16, 16)]
                idx_ref[pl.ds((ch * 8 + c) * npt + g * 16, 16)] = v + cb
        return carry

    lax.fori_loop(0, npt // 16, idx_body, 0)
    pltpu.sync_copy(vol_hbm.at[idx_ref], val_ref)

    def acc_body(g, carry):
        ds = pl.ds(g * 16, 16)
        ws = [wgt[c, ds] for c in range(8)]
        for ch in range(nch):
            acc = jnp.zeros((16,), F32)
            for c in range(8):
                v = val_ref[pl.ds((ch * 8 + c) * npt + g * 16, 16)]
                acc = acc + v * ws[c]
            featv[row_base + ch, ds] = acc
        return carry

    lax.fori_loop(0, npt // 16, acc_body, 0)


@functools.partial(
    pl.kernel,
    mesh=_MESH,
    out_type=jax.ShapeDtypeStruct((56, B * NPTS), F32),
    scratch_types=[
        pltpu.VMEM((256,), F32),
        pltpu.VMEM((256,), F32),
        pltpu.VMEM((256,), F32),
        pltpu.VMEM((8, 256), I32),
        pltpu.VMEM((8, 256), F32),
        pltpu.VMEM((12 * 8 * 256,), I32),
        pltpu.VMEM((12 * 8 * 256,), F32),
        pltpu.VMEM((3 * 8 * 256,), I32),
        pltpu.VMEM((3 * 8 * 256,), F32),
        pltpu.VMEM((56, 256), F32),
    ],
)
def _sc_feat_a(out_hbm, p1_hbm, px_hbm, py_hbm, pz_hbm, feat_hbm, crx, cry,
               crz, vox, wgt, gidx, vals, gidx3, vals3, featv):
    b, s, npt = _feat_common(px_hbm, py_hbm, pz_hbm, crx, cry, crz, vox, wgt)

    _feat_chunk(out_hbm, vox, wgt, featv, b * 3, 3, 0, gidx3, vals3, npt)

    def fine_body(k, carry):
        _feat_chunk(p1_hbm, vox, wgt, featv, b * 48 + k * 12, 12, 3 + k * 12,
                    gidx, vals, npt)
        return carry

    lax.fori_loop(0, 4, fine_body, 0)

    def zero_body(g, carry):
        z = jnp.zeros((16,), F32)
        for r in range(51, 56):
            featv[r, pl.ds(g * 16, 16)] = z
        return carry

    lax.fori_loop(0, npt // 16, zero_body, 0)

    col0 = b * NPTS + s * npt
    pltpu.sync_copy(featv, feat_hbm.at[:, pl.ds(col0, npt)])


@functools.partial(
    pl.kernel,
    mesh=_MESH,
    out_type=jax.ShapeDtypeStruct((48, B * NPTS), F32),
    scratch_types=[
        pltpu.VMEM((256,), F32),
        pltpu.VMEM((256,), F32),
        pltpu.VMEM((256,), F32),
        pltpu.VMEM((8, 256), I32),
        pltpu.VMEM((8, 256), F32),
        pltpu.VMEM((12 * 8 * 256,), I32),
        pltpu.VMEM((12 * 8 * 256,), F32),
        pltpu.VMEM((48, 256), F32),
    ],
)
def _sc_feat_b(p2_hbm, px_hbm, py_hbm, pz_hbm, feat_hbm, crx, cry, crz, vox,
               wgt, gidx, vals, featv):
    b, s, npt = _feat_common(px_hbm, py_hbm, pz_hbm, crx, cry, crz, vox, wgt)

    def fine_body(k, carry):
        _feat_chunk(p2_hbm, vox, wgt, featv, b * 48 + k * 12, 12, k * 12,
                    gidx, vals, npt)
        return carry

    lax.fori_loop(0, 4, fine_body, 0)

    col0 = b * NPTS + s * npt
    pltpu.sync_copy(featv, feat_hbm.at[:, pl.ds(col0, npt)])


def _mlp_body(fa_ref, fb_ref, w1a_ref, w1b_ref, b1_ref, w2_ref, b2_ref,
              w3_ref, b3_ref, o_ref):
    h = (jnp.dot(w1a_ref[...], fa_ref[...], preferred_element_type=F32)
         + jnp.dot(w1b_ref[...], fb_ref[...], preferred_element_type=F32)
         + b1_ref[...])
    h = jnp.maximum(h, 0.0)
    h = jnp.dot(w2_ref[...], h, preferred_element_type=F32) + b2_ref[...]
    h = jnp.maximum(h, 0.0)
    o_ref[...] = jnp.dot(w3_ref[...], h, preferred_element_type=F32) + b3_ref[...]


def _mlp(fa, fb, W1a, W1b, b1c, W2, b2c, W3p, b3c):
    return pl.pallas_call(
        _mlp_body,
        out_shape=jax.ShapeDtypeStruct((8, B * NPTS), F32),
    )(fa, fb, W1a, W1b, b1c, W2, b2c, W3p, b3c)


def kernel(x, p2_1, p2_2, out, W1, b1, W2, b2, W3, b3):
    del x
    key = jax.random.key(42)
    k1, k2 = jax.random.split(key)
    over = jax.random.uniform(k1, (B, KN, 3), dtype=out.dtype)
    coverage = jax.random.uniform(k2, (B, NPTS - BN, 3), dtype=out.dtype)

    out_flat = out.reshape(B * 3 * PLANE)
    p1_flat = p2_1.reshape(B * 48 * PLANE)
    p2_flat = p2_2.reshape(B * 48 * PLANE)

    px, py, pz = _sc_select(
        out_flat,
        over[:, :, 0].reshape(B * KN), over[:, :, 1].reshape(B * KN),
        over[:, :, 2].reshape(B * KN),
        coverage[:, :, 0].reshape(-1), coverage[:, :, 1].reshape(-1),
        coverage[:, :, 2].reshape(-1))
    points = jnp.stack([px, py, pz], axis=-1)

    feat_a = _sc_feat_a(out_flat, p1_flat, px, py, pz)
    feat_b = _sc_feat_b(p2_flat, px, py, pz)

    W1a = jnp.pad(W1[:, :51], ((0, 0), (0, 5)))
    W1b = W1[:, 51:99]
    W3p = jnp.pad(W3, ((0, 5), (0, 0)))
    y = _mlp(feat_a, feat_b, W1a, W1b, b1.reshape(64, 1), W2,
             b2.reshape(32, 1), W3p, jnp.pad(b3, (0, 5)).reshape(8, 1))
    rend = y[:3].reshape(3, B, NPTS).transpose(1, 0, 2)
    return rend, points
